# baseline (speedup 1.0000x reference)
import jax
import jax.numpy as jnp
from jax.experimental import pallas as pl
from jax.experimental.pallas import tpu as pltpu

T, B, N_INP, N_HID, N_OUT = 1024, 64, 128, 512, 128
DT, GAMMA, EPSILON = 0.01, 1.0, 1.0

TCH = 64
NCORE = 2
BH = B // NCORE


def _cornn_kernel(x_ref, wit_ref, bi_ref, wcat_ref, wrot_ref, bro_ref,
                  out_ref, hy_ref, hz_ref, act_ref,
                  hstate, iext):
    t_blk = pl.program_id(1)

    @pl.when(t_blk == 0)
    def _():
        hstate[...] = jnp.zeros_like(hstate)

    xb = x_ref[...].reshape(TCH * BH, N_INP)
    iext[...] = (
        jnp.dot(xb, wit_ref[...], preferred_element_type=jnp.float32)
        + bi_ref[...]
    ).reshape(TCH, BH, N_HID)

    def step(t, carry):
        hs = hstate[...]
        pre = jnp.dot(hs.astype(jnp.bfloat16), wcat_ref[...],
                      preferred_element_type=jnp.float32)
        a = jnp.tanh(pre + iext[t])
        hy = hs[:, :N_HID]
        hz = hs[:, N_HID:]
        hz = hz + DT * (a - GAMMA * hy - EPSILON * hz)
        hy = hy + DT * hz
        act_ref[t] = a
        hy_ref[t] = hy
        hz_ref[t] = hz
        hstate[:, :N_HID] = hy
        hstate[:, N_HID:] = hz
        return carry

    jax.lax.fori_loop(0, TCH, step, 0)

    hys = hy_ref[...].reshape(TCH * BH, N_HID)
    out_ref[...] = (
        jnp.dot(hys, wrot_ref[...], preferred_element_type=jnp.float32)
        + bro_ref[...]
    ).reshape(TCH, BH, N_OUT)


def kernel(x, W_I, b_I, W_R, W_F, W_ro, b_ro):
    wit = W_I.T
    wcat = jnp.concatenate([W_R, W_F], axis=1).T.astype(jnp.bfloat16)
    wrot = W_ro.T
    bi = b_I.reshape(1, N_HID)
    bro = b_ro.reshape(1, N_OUT)

    f32 = jnp.float32
    out_shapes = (
        jax.ShapeDtypeStruct((T, B, N_OUT), f32),
        jax.ShapeDtypeStruct((T, B, N_HID), f32),
        jax.ShapeDtypeStruct((T, B, N_HID), f32),
        jax.ShapeDtypeStruct((T, B, N_HID), f32),
    )
    grid = (NCORE, T // TCH)

    full = lambda shape: pl.BlockSpec(shape, lambda c, t: (0, 0))
    chunk = lambda last: pl.BlockSpec((TCH, BH, last), lambda c, t: (t, c, 0))

    out, hy, hz, act = pl.pallas_call(
        _cornn_kernel,
        out_shape=out_shapes,
        grid=grid,
        in_specs=[
            chunk(N_INP),
            full((N_INP, N_HID)),
            full((1, N_HID)),
            full((2 * N_HID, N_HID)),
            full((N_HID, N_OUT)),
            full((1, N_OUT)),
        ],
        out_specs=(
            chunk(N_OUT),
            chunk(N_HID),
            chunk(N_HID),
            chunk(N_HID),
        ),
        scratch_shapes=[
            pltpu.VMEM((BH, 2 * N_HID), f32),
            pltpu.VMEM((TCH, BH, N_HID), f32),
        ],
        compiler_params=pltpu.CompilerParams(
            dimension_semantics=("parallel", "arbitrary"),
            vmem_limit_bytes=50 * 1024 * 1024,
        ),
        name="cornn_fused",
    )(x, wit, bi, wcat, wrot, bro)
    return out, hy, hz, act


# speedup vs baseline: 1.8591x; 1.8591x over previous
import jax
import jax.numpy as jnp
from jax.experimental import pallas as pl
from jax.experimental.pallas import tpu as pltpu

T, B, N_INP, N_HID, N_OUT = 1024, 64, 128, 512, 128
DT, GAMMA, EPSILON = 0.01, 1.0, 1.0

TCH = 32


def _cornn_kernel(x_ref, wit_ref, bi_ref, wcat_ref, wrot_ref, bro_ref,
                  out_ref, hy_ref, hz_ref, act_ref,
                  hstate, iext):
    t_blk = pl.program_id(0)

    @pl.when(t_blk == 0)
    def _():
        hstate[...] = jnp.zeros_like(hstate)

    xb = x_ref[...].reshape(TCH * B, N_INP)
    iext[...] = (
        jnp.dot(xb, wit_ref[...], preferred_element_type=jnp.float32)
        + bi_ref[...]
    ).reshape(TCH, B, N_HID)

    def step(t, carry):
        hs = hstate[...]
        pre = jnp.dot(hs.astype(jnp.bfloat16), wcat_ref[...],
                      preferred_element_type=jnp.float32)
        a = jnp.tanh(pre + iext[t])
        hy = hs[:, :N_HID]
        hz = hs[:, N_HID:]
        hz = hz + DT * (a - GAMMA * hy - EPSILON * hz)
        hy = hy + DT * hz
        act_ref[t] = a
        hy_ref[t] = hy
        hz_ref[t] = hz
        hstate[:, :N_HID] = hy
        hstate[:, N_HID:] = hz
        return carry

    jax.lax.fori_loop(0, TCH, step, 0, unroll=4)

    hys = hy_ref[...].reshape(TCH * B, N_HID)
    out_ref[...] = (
        jnp.dot(hys, wrot_ref[...], preferred_element_type=jnp.float32)
        + bro_ref[...]
    ).reshape(TCH, B, N_OUT)


def kernel(x, W_I, b_I, W_R, W_F, W_ro, b_ro):
    wit = W_I.T
    wcat = jnp.concatenate([W_R, W_F], axis=1).T.astype(jnp.bfloat16)
    wrot = W_ro.T
    bi = b_I.reshape(1, N_HID)
    bro = b_ro.reshape(1, N_OUT)

    f32 = jnp.float32
    out_shapes = (
        jax.ShapeDtypeStruct((T, B, N_OUT), f32),
        jax.ShapeDtypeStruct((T, B, N_HID), f32),
        jax.ShapeDtypeStruct((T, B, N_HID), f32),
        jax.ShapeDtypeStruct((T, B, N_HID), f32),
    )
    grid = (T // TCH,)

    full = lambda shape: pl.BlockSpec(shape, lambda t: (0, 0))
    chunk = lambda last: pl.BlockSpec((TCH, B, last), lambda t: (t, 0, 0))

    out, hy, hz, act = pl.pallas_call(
        _cornn_kernel,
        out_shape=out_shapes,
        grid=grid,
        in_specs=[
            chunk(N_INP),
            full((N_INP, N_HID)),
            full((1, N_HID)),
            full((2 * N_HID, N_HID)),
            full((N_HID, N_OUT)),
            full((1, N_OUT)),
        ],
        out_specs=(
            chunk(N_OUT),
            chunk(N_HID),
            chunk(N_HID),
            chunk(N_HID),
        ),
        scratch_shapes=[
            pltpu.VMEM((B, 2 * N_HID), f32),
            pltpu.VMEM((TCH, B, N_HID), f32),
        ],
        compiler_params=pltpu.CompilerParams(
            dimension_semantics=("arbitrary",),
            vmem_limit_bytes=50 * 1024 * 1024,
        ),
        name="cornn_fused",
    )(x, wit, bi, wcat, wrot, bro)
    return out, hy, hz, act


# speedup vs baseline: 1.9033x; 1.0238x over previous
import jax
import jax.numpy as jnp
from jax.experimental import pallas as pl
from jax.experimental.pallas import tpu as pltpu

T, B, N_INP, N_HID, N_OUT = 1024, 64, 128, 512, 128
DT, GAMMA, EPSILON = 0.01, 1.0, 1.0

TCH = 32


def _cornn_kernel(x_ref, wit_ref, bi_ref, wcat_ref, wrot_ref, bro_ref,
                  out_ref, hy_ref, hz_ref, act_ref,
                  hstate, iext):
    t_blk = pl.program_id(0)

    @pl.when(t_blk == 0)
    def _():
        hstate[...] = jnp.zeros_like(hstate)

    xb = x_ref[...].reshape(TCH * B, N_INP)
    iext[...] = (
        jnp.dot(xb, wit_ref[...], preferred_element_type=jnp.float32)
        + bi_ref[...]
    ).reshape(TCH, B, N_HID)

    def step(t, carry):
        hs = hstate[...]
        pre = jnp.dot(hs.astype(jnp.bfloat16), wcat_ref[...],
                      preferred_element_type=jnp.float32)
        a = jnp.tanh(pre + iext[t])
        hy = hs[:, :N_HID]
        hz = hs[:, N_HID:]
        hz = hz + DT * (a - GAMMA * hy - EPSILON * hz)
        hy = hy + DT * hz
        act_ref[t] = a
        hy_ref[t] = hy
        hz_ref[t] = hz
        hstate[:, :N_HID] = hy
        hstate[:, N_HID:] = hz
        return carry

    jax.lax.fori_loop(0, TCH, step, 0, unroll=8)

    hys = hy_ref[...].reshape(TCH * B, N_HID)
    out_ref[...] = (
        jnp.dot(hys, wrot_ref[...], preferred_element_type=jnp.float32)
        + bro_ref[...]
    ).reshape(TCH, B, N_OUT)


def kernel(x, W_I, b_I, W_R, W_F, W_ro, b_ro):
    wit = W_I.T
    wcat = jnp.concatenate([W_R, W_F], axis=1).T.astype(jnp.bfloat16)
    wrot = W_ro.T
    bi = b_I.reshape(1, N_HID)
    bro = b_ro.reshape(1, N_OUT)

    f32 = jnp.float32
    out_shapes = (
        jax.ShapeDtypeStruct((T, B, N_OUT), f32),
        jax.ShapeDtypeStruct((T, B, N_HID), f32),
        jax.ShapeDtypeStruct((T, B, N_HID), f32),
        jax.ShapeDtypeStruct((T, B, N_HID), f32),
    )
    grid = (T // TCH,)

    full = lambda shape: pl.BlockSpec(shape, lambda t: (0, 0))
    chunk = lambda last: pl.BlockSpec((TCH, B, last), lambda t: (t, 0, 0))

    out, hy, hz, act = pl.pallas_call(
        _cornn_kernel,
        out_shape=out_shapes,
        grid=grid,
        in_specs=[
            chunk(N_INP),
            full((N_INP, N_HID)),
            full((1, N_HID)),
            full((2 * N_HID, N_HID)),
            full((N_HID, N_OUT)),
            full((1, N_OUT)),
        ],
        out_specs=(
            chunk(N_OUT),
            chunk(N_HID),
            chunk(N_HID),
            chunk(N_HID),
        ),
        scratch_shapes=[
            pltpu.VMEM((B, 2 * N_HID), f32),
            pltpu.VMEM((TCH, B, N_HID), f32),
        ],
        compiler_params=pltpu.CompilerParams(
            dimension_semantics=("arbitrary",),
            vmem_limit_bytes=50 * 1024 * 1024,
        ),
        name="cornn_fused",
    )(x, wit, bi, wcat, wrot, bro)
    return out, hy, hz, act
